# Initial kernel scaffold; baseline (speedup 1.0000x reference)
import jax
import jax.numpy as jnp
from jax.experimental import pallas as pl

EPS_LN = 1e-12


def kernel(input_ids, token_type_ids, word_emb, pos_emb, type_emb, ln_weight, ln_bias):
    B, S = input_ids.shape
    H = word_emb.shape[1]
    embeds = jnp.take(word_emb, input_ids, axis=0)
    tt3 = token_type_ids.reshape(B, 1, S)

    def body(emb_ref, tt_ref, pos_ref, type_ref, w_ref, b_ref, out_ref):
        x = emb_ref[0]
        tt = tt_ref[0, 0]
        t0 = type_ref[0]
        t1 = type_ref[1]
        tmask = (tt == 1)[:, None]
        x = x + pos_ref[...] + jnp.where(tmask, t1[None, :], t0[None, :])
        mean = jnp.mean(x, axis=-1, keepdims=True)
        xc = x - mean
        var = jnp.mean(xc * xc, axis=-1, keepdims=True)
        y = xc * jax.lax.rsqrt(var + EPS_LN)
        out_ref[0] = y * w_ref[...][None, :] + b_ref[...][None, :]

    out = pl.pallas_call(
        body,
        grid=(B,),
        in_specs=[
            pl.BlockSpec((1, S, H), lambda i: (i, 0, 0)),
            pl.BlockSpec((1, 1, S), lambda i: (i, 0, 0)),
            pl.BlockSpec((S, H), lambda i: (0, 0)),
            pl.BlockSpec((2, H), lambda i: (0, 0)),
            pl.BlockSpec((H,), lambda i: (0,)),
            pl.BlockSpec((H,), lambda i: (0,)),
        ],
        out_specs=pl.BlockSpec((1, S, H), lambda i: (i, 0, 0)),
        out_shape=jax.ShapeDtypeStruct((B, S, H), jnp.float32),
    )(embeds, tt3, pos_emb, type_emb, ln_weight, ln_bias)
    return out


# speedup vs baseline: 1.5098x; 1.5098x over previous
import jax
import jax.numpy as jnp
from jax.experimental import pallas as pl

EPS_LN = 1e-12


def kernel(input_ids, token_type_ids, word_emb, pos_emb, type_emb, ln_weight, ln_bias):
    B, S = input_ids.shape
    H = word_emb.shape[1]
    embeds = jnp.take(word_emb, input_ids, axis=0)
    tt3 = token_type_ids.reshape(B, S, 1)

    def body(emb_ref, tt_ref, pos_ref, type_ref, w_ref, b_ref, out_ref):
        x = emb_ref[0]
        tt = tt_ref[0]
        t0 = type_ref[0]
        t1 = type_ref[1]
        tmask = tt == 1
        x = x + pos_ref[...] + jnp.where(tmask, t1[None, :], t0[None, :])
        mean = jnp.mean(x, axis=-1, keepdims=True)
        xc = x - mean
        var = jnp.mean(xc * xc, axis=-1, keepdims=True)
        y = xc * jax.lax.rsqrt(var + EPS_LN)
        out_ref[0] = y * w_ref[...][None, :] + b_ref[...][None, :]

    out = pl.pallas_call(
        body,
        grid=(B,),
        in_specs=[
            pl.BlockSpec((1, S, H), lambda i: (i, 0, 0)),
            pl.BlockSpec((1, S, 1), lambda i: (i, 0, 0)),
            pl.BlockSpec((S, H), lambda i: (0, 0)),
            pl.BlockSpec((2, H), lambda i: (0, 0)),
            pl.BlockSpec((H,), lambda i: (0,)),
            pl.BlockSpec((H,), lambda i: (0,)),
        ],
        out_specs=pl.BlockSpec((1, S, H), lambda i: (i, 0, 0)),
        out_shape=jax.ShapeDtypeStruct((B, S, H), jnp.float32),
    )(embeds, tt3, pos_emb, type_emb, ln_weight, ln_bias)
    return out
